# baseline (speedup 1.0000x reference)
import functools

import jax
import jax.numpy as jnp
from jax import lax
from jax.experimental import pallas as pl
from jax.experimental.pallas import tpu as pltpu
from jax.experimental.pallas import tpu_sc as plsc

NUM_CLASSES = 1000
HIDDEN = 1152
BATCH = 16384
UNCOND_ID = NUM_CLASSES

NC = 2
NS = 16
L = 16
NW = NC * NS
B_PER_W = BATCH // NW
CHUNK = 32
NCHUNK = B_PER_W // CHUNK


def _make_kernel():
    mesh = plsc.VectorSubcoreMesh(core_axis_name="c", subcore_axis_name="s")

    @functools.partial(
        pl.kernel,
        mesh=mesh,
        out_type=jax.ShapeDtypeStruct((BATCH, HIDDEN), jnp.float32),
        scratch_types=[
            pltpu.VMEM((B_PER_W,), jnp.int32),
            pltpu.VMEM((B_PER_W,), jnp.int32),
            pltpu.VMEM((B_PER_W,), jnp.int32),
            pltpu.VMEM((CHUNK, HIDDEN), jnp.float32),
            pltpu.VMEM((CHUNK, HIDDEN), jnp.float32),
            pltpu.SemaphoreType.DMA,
            pltpu.SemaphoreType.DMA,
            pltpu.SemaphoreType.DMA,
            pltpu.SemaphoreType.DMA,
        ],
    )
    def emb_kernel(labels_hbm, drop_hbm, table_hbm, out_hbm,
                   lab_v, drop_v, idx_v, buf0, buf1, g0, g1, s0, s1):
        wid = lax.axis_index("s") * NC + lax.axis_index("c")
        base = wid * B_PER_W

        pltpu.sync_copy(labels_hbm.at[pl.ds(base, B_PER_W)], lab_v)
        pltpu.sync_copy(drop_hbm.at[pl.ds(base, B_PER_W)], drop_v)

        def sel_body(i, carry):
            off = i * L
            lab = lab_v[pl.ds(off, L)]
            dr = drop_v[pl.ds(off, L)]
            idx_v[pl.ds(off, L)] = jnp.where(
                dr != 0, jnp.full((L,), UNCOND_ID, jnp.int32), lab)
            return carry

        lax.fori_loop(0, B_PER_W // L, sel_body, 0)

        bufs = (buf0, buf1)
        gsem = (g0, g1)
        ssem = (s0, s1)

        def gath(c, slot):
            return pltpu.make_async_copy(
                table_hbm.at[idx_v.at[pl.ds(c * CHUNK, CHUNK)]],
                bufs[slot], gsem[slot])

        def scat(c, slot):
            return pltpu.make_async_copy(
                bufs[slot], out_hbm.at[pl.ds(base + c * CHUNK, CHUNK)],
                ssem[slot])

        gath(0, 0).start()
        for c in range(NCHUNK):
            slot = c & 1
            if c + 1 < NCHUNK:
                if c >= 1:
                    scat(c - 1, slot ^ 1).wait()
                gath(c + 1, slot ^ 1).start()
            gath(c, slot).wait()
            scat(c, slot).start()
        scat(NCHUNK - 2, 0).wait()
        scat(NCHUNK - 1, 1).wait()

    return emb_kernel


_emb_kernel = _make_kernel()


def kernel(labels, train, force_drop_ids, table):
    del train
    return _emb_kernel(labels.astype(jnp.int32),
                       force_drop_ids.astype(jnp.int32),
                       table)
